# baseline (speedup 1.0000x reference)
import jax
import jax.numpy as jnp
from jax import lax
from jax.experimental import pallas as pl
from jax.experimental.pallas import tpu as pltpu
from jax.experimental.pallas import tpu_sc as plsc

N_PAD = 10240
NC, NS = 2, 16
NW = NC * NS
CH = 128
ROWS_PER_TILE = N_PAD // NS
NBUF = 12
LAG = 6

_SC_PARAMS = pltpu.CompilerParams(
    needs_layout_passes=False, use_tc_tiling_on_sc=False)


def _chunks(epw):
    out = []
    off = 0
    while off < epw:
        n = min(CH, epw - off)
        out.append((off, n))
        off += n
    return out


def _deg_kernel_body(ei_hbm, zeros1_hbm, degp_hbm, dstv, deg_a, deg_b, lsem):
    c = lax.axis_index("c")
    s = lax.axis_index("s")
    gid = c * NS + s
    epw = dstv.shape[0]
    npad = deg_a.shape[0]
    e_total = ei_hbm.shape[0] // 2

    ld0 = pltpu.async_copy(zeros1_hbm, deg_a, lsem)
    ld1 = pltpu.async_copy(
        ei_hbm.at[pl.ds(e_total + gid * epw, epw)], dstv, lsem)
    ld2 = pltpu.async_copy(zeros1_hbm, deg_b, lsem)
    ld0.wait()
    ld1.wait()
    ld2.wait()

    ones16 = jnp.full((16,), 1.0, jnp.float32)

    def hist_body(r, carry):
        for u in range(4):
            idx = dstv[pl.ds((r * 4 + u) * 16, 16)]
            plsc.addupdate_scatter(deg_a if u % 2 == 0 else deg_b,
                                   [idx], ones16)
        return carry

    lax.fori_loop(0, epw // 64, hist_body, 0)
    for t in range(epw // 64 * 4, epw // 16):
        plsc.addupdate_scatter(deg_a, [dstv[pl.ds(t * 16, 16)]], ones16)

    def merge_body(r, carry):
        for u in range(4):
            sl = pl.ds((r * 4 + u) * 16, 16)
            plsc.addupdate(deg_a.at[sl], deg_b[sl])
        return carry

    lax.fori_loop(0, npad // 64, merge_body, 0)

    pltpu.sync_copy(deg_a, degp_hbm.at[gid])


def _msg_kernel_body(ei_hbm, h2_hbm, dis_hbm, b_hbm, iota_hbm,
                     zeros2_hbm, out_hbm, srcv, dstv, rows, selfv, iotav,
                     disv, bv, obuf, acc, lsem, gsem, ssem):
    c = lax.axis_index("c")
    s = lax.axis_index("s")
    gid = c * NS + s
    epw = srcv.shape[0]
    e_total = ei_hbm.shape[0] // 2
    chunks = _chunks(epw)
    sl = pl.ds(s * ROWS_PER_TILE, ROWS_PER_TILE)

    ld0 = pltpu.async_copy(zeros2_hbm.at[sl], acc.at[sl], lsem)
    ld1 = pltpu.async_copy(ei_hbm.at[pl.ds(gid * epw, epw)], srcv, lsem)
    ld2 = pltpu.async_copy(
        ei_hbm.at[pl.ds(e_total + gid * epw, epw)], dstv, lsem)
    ld3 = pltpu.async_copy(dis_hbm.at[sl], disv, lsem)
    ld4 = pltpu.async_copy(b_hbm, bv, lsem)
    ld5 = pltpu.async_copy(iota_hbm.at[sl], iotav, lsem)
    ld6 = pltpu.async_copy(h2_hbm.at[sl], selfv, lsem)
    ld0.wait()
    ld1.wait()
    ld2.wait()
    ld3.wait()
    ld4.wait()
    ld5.wait()
    ld6.wait()
    plsc.subcore_barrier()

    n_chunks = len(chunks)
    gd = {}
    sd = {}
    for j in range(n_chunks + LAG):
        if j < n_chunks:
            off, n = chunks[j]
            if j >= NBUF:
                sd.pop(j - NBUF).wait()
            gd[j] = pltpu.async_copy(
                h2_hbm.at[srcv.at[pl.ds(off, n)]],
                rows.at[j % NBUF, pl.ds(0, n)], gsem)
        jj = j - LAG
        if jj >= 0:
            off, n = chunks[jj]
            gd.pop(jj).wait()
            sd[jj] = pltpu.async_copy(
                rows.at[jj % NBUF, pl.ds(0, n)],
                acc.at[dstv.at[pl.ds(off, n)]], ssem, add=True)
    for jj in sorted(sd):
        sd.pop(jj).wait()

    @pl.when(c == 0)
    def _inject():
        for k in range(ROWS_PER_TILE // CH):
            pltpu.sync_copy(
                selfv.at[pl.ds(k * CH, CH)],
                acc.at[iotav.at[pl.ds(k * CH, CH)]], add=True)

    plsc.subcore_barrier()

    pltpu.sync_copy(acc.at[sl], obuf)
    wb = jnp.where(c == 0, 1.0, 0.0)
    b_eff = bv[...] * wb

    def scale_body(g, carry):
        disg = disv[pl.ds(g * 16, 16)]
        for r in range(16):
            onehot = (lax.iota(jnp.int32, 16) == r).astype(jnp.float32)
            sr = jnp.sum(disg * onehot, axis=0)
            row = obuf[g * 16 + r]
            obuf[g * 16 + r] = row * sr + b_eff
        return carry

    lax.fori_loop(0, ROWS_PER_TILE // 16, scale_body, 0)
    pltpu.sync_copy(obuf, out_hbm.at[c, sl])


def _matmul_body(x_ref, w_ref, ht_ref):
    ht_ref[...] = jnp.dot(w_ref[...].T, x_ref[...].T,
                          preferred_element_type=jnp.float32)


def _prep_body(ht_ref, degp_ref, h2t_ref, dis_ref):
    n = ht_ref.shape[1]
    npad = h2t_ref.shape[1]
    deg = jnp.sum(degp_ref[...], axis=0) + 1.0
    dis = lax.rsqrt(deg)
    h2t_ref[:, pl.ds(0, n)] = ht_ref[...] * dis[None, :n]
    h2t_ref[:, pl.ds(n, npad - n)] = jnp.zeros(
        (ht_ref.shape[0], npad - n), jnp.float32)
    dis_ref[...] = dis


def _final_body(acc_ref, out_ref):
    nf = out_ref.shape[0]
    half = acc_ref.shape[0] // 2
    out_ref[...] = acc_ref[pl.ds(0, nf)] + acc_ref[pl.ds(half, nf)]


def kernel(x, edge_index, W, b):
    N, F = x.shape
    C = W.shape[1]
    E = edge_index.shape[1]

    ei = edge_index.astype(jnp.int32).reshape(-1)
    epw = E // NW
    zeros1 = jnp.zeros((N_PAD,), jnp.float32)
    zeros2 = jnp.zeros((N_PAD, C), jnp.float32)
    iota = jnp.arange(N_PAD, dtype=jnp.int32)

    mesh = plsc.VectorSubcoreMesh(core_axis_name="c", subcore_axis_name="s")

    ht = pl.pallas_call(
        _matmul_body,
        out_shape=jax.ShapeDtypeStruct((C, N), jnp.float32),
    )(x, W)

    deg_kernel = pl.kernel(
        _deg_kernel_body,
        out_type=jax.ShapeDtypeStruct((NW, N_PAD), jnp.float32),
        mesh=mesh,
        scratch_types=[
            pltpu.VMEM((epw,), jnp.int32),
            pltpu.VMEM((N_PAD,), jnp.float32),
            pltpu.VMEM((N_PAD,), jnp.float32),
            pltpu.SemaphoreType.DMA,
        ],
        compiler_params=_SC_PARAMS,
    )
    degp = deg_kernel(ei, zeros1)

    h2t, dis = pl.pallas_call(
        _prep_body,
        out_shape=[
            jax.ShapeDtypeStruct((C, N_PAD), jnp.float32),
            jax.ShapeDtypeStruct((N_PAD,), jnp.float32),
        ],
    )(ht, degp)
    h2 = h2t.T

    msg_kernel = pl.kernel(
        _msg_kernel_body,
        out_type=jax.ShapeDtypeStruct((NC, N_PAD, C), jnp.float32),
        mesh=mesh,
        scratch_types=[
            pltpu.VMEM((epw,), jnp.int32),
            pltpu.VMEM((epw,), jnp.int32),
            pltpu.VMEM((NBUF, CH, C), jnp.float32),
            pltpu.VMEM((ROWS_PER_TILE, C), jnp.float32),
            pltpu.VMEM((ROWS_PER_TILE,), jnp.int32),
            pltpu.VMEM((ROWS_PER_TILE,), jnp.float32),
            pltpu.VMEM((C,), jnp.float32),
            pltpu.VMEM((ROWS_PER_TILE, C), jnp.float32),
            pltpu.VMEM_SHARED((N_PAD, C), jnp.float32),
            pltpu.SemaphoreType.DMA,
            pltpu.SemaphoreType.DMA,
            pltpu.SemaphoreType.DMA,
        ],
        compiler_params=_SC_PARAMS,
    )
    accp = msg_kernel(ei, h2, dis, b, iota, zeros2)

    out1d = pl.pallas_call(
        _final_body,
        out_shape=jax.ShapeDtypeStruct((N * C,), jnp.float32),
    )(accp.reshape(NC * N_PAD * C))

    return out1d.reshape(N, C)


# speedup vs baseline: 1.0468x; 1.0468x over previous
import jax
import jax.numpy as jnp
from jax import lax
from jax.experimental import pallas as pl
from jax.experimental.pallas import tpu as pltpu
from jax.experimental.pallas import tpu_sc as plsc

N_PAD = 10240
NC, NS = 2, 16
NW = NC * NS
CH = 128
ROWS_PER_TILE = N_PAD // NS
NBUF = 12
LAG = 6

_SC_PARAMS = pltpu.CompilerParams(
    needs_layout_passes=False, use_tc_tiling_on_sc=False)


def _chunks(epw):
    out = []
    off = 0
    while off < epw:
        n = min(CH, epw - off)
        out.append((off, n))
        off += n
    return out


def _deg_kernel_body(ei_hbm, zeros1_hbm, degp_hbm, dstv, deg_local, lsem):
    c = lax.axis_index("c")
    s = lax.axis_index("s")
    gid = c * NS + s
    epw = dstv.shape[0]
    e_total = ei_hbm.shape[0] // 2

    ld0 = pltpu.async_copy(zeros1_hbm, deg_local, lsem)
    ld1 = pltpu.async_copy(
        ei_hbm.at[pl.ds(e_total + gid * epw, epw)], dstv, lsem)
    ld0.wait()
    ld1.wait()

    ones16 = jnp.full((16,), 1.0, jnp.float32)

    def hist_body(r, carry):
        for u in range(5):
            idx = dstv[pl.ds((r * 5 + u) * 16, 16)]
            plsc.addupdate_scatter(deg_local, [idx], ones16)
        return carry

    lax.fori_loop(0, epw // 80, hist_body, 0)

    pltpu.sync_copy(deg_local, degp_hbm.at[gid])


def _msg_kernel_body(ei_hbm, h2_hbm, dis_hbm, b_hbm, iota_hbm,
                     zeros2_hbm, out_hbm, srcv, dstv, rows, selfv, iotav,
                     disv, bv, obuf, acc, lsem, gsem, ssem):
    c = lax.axis_index("c")
    s = lax.axis_index("s")
    gid = c * NS + s
    epw = srcv.shape[0]
    e_total = ei_hbm.shape[0] // 2
    chunks = _chunks(epw)
    sl = pl.ds(s * ROWS_PER_TILE, ROWS_PER_TILE)

    ld0 = pltpu.async_copy(zeros2_hbm.at[sl], acc.at[sl], lsem)
    ld1 = pltpu.async_copy(ei_hbm.at[pl.ds(gid * epw, epw)], srcv, lsem)
    ld2 = pltpu.async_copy(
        ei_hbm.at[pl.ds(e_total + gid * epw, epw)], dstv, lsem)
    ld3 = pltpu.async_copy(dis_hbm.at[sl], disv, lsem)
    ld4 = pltpu.async_copy(b_hbm, bv, lsem)
    ld5 = pltpu.async_copy(iota_hbm.at[sl], iotav, lsem)
    ld6 = pltpu.async_copy(h2_hbm.at[sl], selfv, lsem)
    ld0.wait()
    ld1.wait()
    ld2.wait()
    ld3.wait()
    ld4.wait()
    ld5.wait()
    ld6.wait()
    plsc.subcore_barrier()

    n_chunks = len(chunks)
    gd = {}
    sd = {}
    for j in range(n_chunks + LAG):
        if j < n_chunks:
            off, n = chunks[j]
            if j >= NBUF:
                sd.pop(j - NBUF).wait()
            gd[j] = pltpu.async_copy(
                h2_hbm.at[srcv.at[pl.ds(off, n)]],
                rows.at[j % NBUF, pl.ds(0, n)], gsem)
        jj = j - LAG
        if jj >= 0:
            off, n = chunks[jj]
            gd.pop(jj).wait()
            sd[jj] = pltpu.async_copy(
                rows.at[jj % NBUF, pl.ds(0, n)],
                acc.at[dstv.at[pl.ds(off, n)]], ssem, add=True)
    for jj in sorted(sd):
        sd.pop(jj).wait()

    @pl.when(c == 0)
    def _inject():
        for k in range(ROWS_PER_TILE // CH):
            pltpu.sync_copy(
                selfv.at[pl.ds(k * CH, CH)],
                acc.at[iotav.at[pl.ds(k * CH, CH)]], add=True)

    plsc.subcore_barrier()

    pltpu.sync_copy(acc.at[sl], obuf)
    wb = jnp.where(c == 0, 1.0, 0.0)
    b_eff = bv[...] * wb

    def scale_body(g, carry):
        disg = disv[pl.ds(g * 16, 16)]
        for r in range(16):
            onehot = (lax.iota(jnp.int32, 16) == r).astype(jnp.float32)
            sr = jnp.sum(disg * onehot, axis=0)
            row = obuf[g * 16 + r]
            obuf[g * 16 + r] = row * sr + b_eff
        return carry

    lax.fori_loop(0, ROWS_PER_TILE // 16, scale_body, 0)
    pltpu.sync_copy(obuf, out_hbm.at[c, sl])


def _matmul_body(x_ref, w_ref, ht_ref):
    ht_ref[...] = jnp.dot(w_ref[...].T, x_ref[...].T,
                          preferred_element_type=jnp.float32)


def _prep_body(ht_ref, degp_ref, h2t_ref, dis_ref):
    n = ht_ref.shape[1]
    npad = h2t_ref.shape[1]
    deg = jnp.sum(degp_ref[...], axis=0) + 1.0
    dis = lax.rsqrt(deg)
    h2t_ref[:, pl.ds(0, n)] = ht_ref[...] * dis[None, :n]
    h2t_ref[:, pl.ds(n, npad - n)] = jnp.zeros(
        (ht_ref.shape[0], npad - n), jnp.float32)
    dis_ref[...] = dis


def _final_body(acc_ref, out_ref):
    nf = out_ref.shape[0]
    half = acc_ref.shape[0] // 2
    out_ref[...] = acc_ref[pl.ds(0, nf)] + acc_ref[pl.ds(half, nf)]


def kernel(x, edge_index, W, b):
    N, F = x.shape
    C = W.shape[1]
    E = edge_index.shape[1]

    ei = edge_index.astype(jnp.int32).reshape(-1)
    epw = E // NW
    zeros1 = jnp.zeros((N_PAD,), jnp.float32)
    zeros2 = jnp.zeros((N_PAD, C), jnp.float32)
    iota = jnp.arange(N_PAD, dtype=jnp.int32)

    mesh = plsc.VectorSubcoreMesh(core_axis_name="c", subcore_axis_name="s")

    ht = pl.pallas_call(
        _matmul_body,
        out_shape=jax.ShapeDtypeStruct((C, N), jnp.float32),
    )(x, W)

    deg_kernel = pl.kernel(
        _deg_kernel_body,
        out_type=jax.ShapeDtypeStruct((NW, N_PAD), jnp.float32),
        mesh=mesh,
        scratch_types=[
            pltpu.VMEM((epw,), jnp.int32),
            pltpu.VMEM((N_PAD,), jnp.float32),
            pltpu.SemaphoreType.DMA,
        ],
        compiler_params=_SC_PARAMS,
    )
    degp = deg_kernel(ei, zeros1)

    h2t, dis = pl.pallas_call(
        _prep_body,
        out_shape=[
            jax.ShapeDtypeStruct((C, N_PAD), jnp.float32),
            jax.ShapeDtypeStruct((N_PAD,), jnp.float32),
        ],
    )(ht, degp)
    h2 = h2t.T

    msg_kernel = pl.kernel(
        _msg_kernel_body,
        out_type=jax.ShapeDtypeStruct((NC, N_PAD, C), jnp.float32),
        mesh=mesh,
        scratch_types=[
            pltpu.VMEM((epw,), jnp.int32),
            pltpu.VMEM((epw,), jnp.int32),
            pltpu.VMEM((NBUF, CH, C), jnp.float32),
            pltpu.VMEM((ROWS_PER_TILE, C), jnp.float32),
            pltpu.VMEM((ROWS_PER_TILE,), jnp.int32),
            pltpu.VMEM((ROWS_PER_TILE,), jnp.float32),
            pltpu.VMEM((C,), jnp.float32),
            pltpu.VMEM((ROWS_PER_TILE, C), jnp.float32),
            pltpu.VMEM_SHARED((N_PAD, C), jnp.float32),
            pltpu.SemaphoreType.DMA,
            pltpu.SemaphoreType.DMA,
            pltpu.SemaphoreType.DMA,
        ],
        compiler_params=_SC_PARAMS,
    )
    accp = msg_kernel(ei, h2, dis, b, iota, zeros2)

    out1d = pl.pallas_call(
        _final_body,
        out_shape=jax.ShapeDtypeStruct((N * C,), jnp.float32),
    )(accp.reshape(NC * N_PAD * C))

    return out1d.reshape(N, C)


# speedup vs baseline: 1.1210x; 1.0709x over previous
import jax
import jax.numpy as jnp
from jax import lax
from jax.experimental import pallas as pl
from jax.experimental.pallas import tpu as pltpu
from jax.experimental.pallas import tpu_sc as plsc

N_PAD = 10240
NC, NS = 2, 16
NW = NC * NS
CH = 128
ROWS_PER_TILE = N_PAD // NS
NBUF = 12
LAG = 6

_SC_PARAMS = pltpu.CompilerParams(
    needs_layout_passes=False, use_tc_tiling_on_sc=False)


def _chunks(epw):
    out = []
    off = 0
    while off < epw:
        n = min(CH, epw - off)
        out.append((off, n))
        off += n
    return out


def _deg_kernel_body(ei_hbm, zeros1_hbm, degp_hbm, dstv, deg_local, lsem):
    c = lax.axis_index("c")
    s = lax.axis_index("s")
    gid = c * NS + s
    epw = dstv.shape[0]
    e_total = ei_hbm.shape[0] // 2

    ld0 = pltpu.async_copy(zeros1_hbm, deg_local, lsem)
    ld1 = pltpu.async_copy(
        ei_hbm.at[pl.ds(e_total + gid * epw, epw)], dstv, lsem)
    ld0.wait()
    ld1.wait()

    ones16 = jnp.full((16,), 1.0, jnp.float32)

    def hist_body(r, carry):
        for u in range(5):
            idx = dstv[pl.ds((r * 5 + u) * 16, 16)]
            plsc.addupdate_scatter(deg_local, [idx], ones16)
        return carry

    lax.fori_loop(0, epw // 80, hist_body, 0)

    pltpu.sync_copy(deg_local, degp_hbm.at[gid])


def _msg_kernel_body(ei_hbm, h2_hbm, dis_hbm, b_hbm, iota_hbm,
                     zeros2_hbm, out_hbm, srcv, dstv, rows, selfv, iotav,
                     disv, bv, obuf, acc, h2s, lsem, gsem, ssem):
    c = lax.axis_index("c")
    s = lax.axis_index("s")
    gid = c * NS + s
    epw = srcv.shape[0]
    e_total = ei_hbm.shape[0] // 2
    chunks = _chunks(epw)
    sl = pl.ds(s * ROWS_PER_TILE, ROWS_PER_TILE)

    ld0 = pltpu.async_copy(zeros2_hbm.at[sl], acc.at[sl], lsem)
    ld1 = pltpu.async_copy(ei_hbm.at[pl.ds(gid * epw, epw)], srcv, lsem)
    ld2 = pltpu.async_copy(
        ei_hbm.at[pl.ds(e_total + gid * epw, epw)], dstv, lsem)
    ld3 = pltpu.async_copy(dis_hbm.at[sl], disv, lsem)
    ld4 = pltpu.async_copy(b_hbm, bv, lsem)
    ld5 = pltpu.async_copy(iota_hbm.at[sl], iotav, lsem)
    ld6 = pltpu.async_copy(h2_hbm.at[sl], selfv, lsem)
    ld0.wait()
    ld1.wait()
    ld2.wait()
    ld3.wait()
    ld4.wait()
    ld5.wait()
    ld6.wait()
    pltpu.sync_copy(selfv, h2s.at[sl])
    plsc.subcore_barrier()

    n_chunks = len(chunks)
    gd = {}
    sd = {}
    for j in range(n_chunks + LAG):
        if j < n_chunks:
            off, n = chunks[j]
            if j >= NBUF:
                sd.pop(j - NBUF).wait()
            gd[j] = pltpu.async_copy(
                h2s.at[srcv.at[pl.ds(off, n)]],
                rows.at[j % NBUF, pl.ds(0, n)], gsem)
        jj = j - LAG
        if jj >= 0:
            off, n = chunks[jj]
            gd.pop(jj).wait()
            sd[jj] = pltpu.async_copy(
                rows.at[jj % NBUF, pl.ds(0, n)],
                acc.at[dstv.at[pl.ds(off, n)]], ssem, add=True)
    for jj in sorted(sd):
        sd.pop(jj).wait()

    @pl.when(c == 0)
    def _inject():
        for k in range(ROWS_PER_TILE // CH):
            pltpu.sync_copy(
                selfv.at[pl.ds(k * CH, CH)],
                acc.at[iotav.at[pl.ds(k * CH, CH)]], add=True)

    plsc.subcore_barrier()

    pltpu.sync_copy(acc.at[sl], obuf)
    wb = jnp.where(c == 0, 1.0, 0.0)
    b_eff = bv[...] * wb

    def scale_body(g, carry):
        disg = disv[pl.ds(g * 16, 16)]
        for r in range(16):
            onehot = (lax.iota(jnp.int32, 16) == r).astype(jnp.float32)
            sr = jnp.sum(disg * onehot, axis=0)
            row = obuf[g * 16 + r]
            obuf[g * 16 + r] = row * sr + b_eff
        return carry

    lax.fori_loop(0, ROWS_PER_TILE // 16, scale_body, 0)
    pltpu.sync_copy(obuf, out_hbm.at[c, sl])


def _matmul_body(x_ref, w_ref, ht_ref):
    ht_ref[...] = jnp.dot(w_ref[...].T, x_ref[...].T,
                          preferred_element_type=jnp.float32)


def _prep_body(ht_ref, degp_ref, h2t_ref, dis_ref):
    n = ht_ref.shape[1]
    npad = h2t_ref.shape[1]
    deg = jnp.sum(degp_ref[...], axis=0) + 1.0
    dis = lax.rsqrt(deg)
    h2t_ref[:, pl.ds(0, n)] = ht_ref[...] * dis[None, :n]
    h2t_ref[:, pl.ds(n, npad - n)] = jnp.zeros(
        (ht_ref.shape[0], npad - n), jnp.float32)
    dis_ref[...] = dis


def _final_body(acc_ref, out_ref):
    nf = out_ref.shape[0]
    half = acc_ref.shape[0] // 2
    out_ref[...] = acc_ref[pl.ds(0, nf)] + acc_ref[pl.ds(half, nf)]


def kernel(x, edge_index, W, b):
    N, F = x.shape
    C = W.shape[1]
    E = edge_index.shape[1]

    ei = edge_index.astype(jnp.int32).reshape(-1)
    epw = E // NW
    zeros1 = jnp.zeros((N_PAD,), jnp.float32)
    zeros2 = jnp.zeros((N_PAD, C), jnp.float32)
    iota = jnp.arange(N_PAD, dtype=jnp.int32)

    mesh = plsc.VectorSubcoreMesh(core_axis_name="c", subcore_axis_name="s")

    ht = pl.pallas_call(
        _matmul_body,
        out_shape=jax.ShapeDtypeStruct((C, N), jnp.float32),
    )(x, W)

    deg_kernel = pl.kernel(
        _deg_kernel_body,
        out_type=jax.ShapeDtypeStruct((NW, N_PAD), jnp.float32),
        mesh=mesh,
        scratch_types=[
            pltpu.VMEM((epw,), jnp.int32),
            pltpu.VMEM((N_PAD,), jnp.float32),
            pltpu.SemaphoreType.DMA,
        ],
        compiler_params=_SC_PARAMS,
    )
    degp = deg_kernel(ei, zeros1)

    h2t, dis = pl.pallas_call(
        _prep_body,
        out_shape=[
            jax.ShapeDtypeStruct((C, N_PAD), jnp.float32),
            jax.ShapeDtypeStruct((N_PAD,), jnp.float32),
        ],
    )(ht, degp)
    h2 = h2t.T

    msg_kernel = pl.kernel(
        _msg_kernel_body,
        out_type=jax.ShapeDtypeStruct((NC, N_PAD, C), jnp.float32),
        mesh=mesh,
        scratch_types=[
            pltpu.VMEM((epw,), jnp.int32),
            pltpu.VMEM((epw,), jnp.int32),
            pltpu.VMEM((NBUF, CH, C), jnp.float32),
            pltpu.VMEM((ROWS_PER_TILE, C), jnp.float32),
            pltpu.VMEM((ROWS_PER_TILE,), jnp.int32),
            pltpu.VMEM((ROWS_PER_TILE,), jnp.float32),
            pltpu.VMEM((C,), jnp.float32),
            pltpu.VMEM((ROWS_PER_TILE, C), jnp.float32),
            pltpu.VMEM_SHARED((N_PAD, C), jnp.float32),
            pltpu.VMEM_SHARED((N_PAD, C), jnp.float32),
            pltpu.SemaphoreType.DMA,
            pltpu.SemaphoreType.DMA,
            pltpu.SemaphoreType.DMA,
        ],
        compiler_params=_SC_PARAMS,
    )
    accp = msg_kernel(ei, h2, dis, b, iota, zeros2)

    out1d = pl.pallas_call(
        _final_body,
        out_shape=jax.ShapeDtypeStruct((N * C,), jnp.float32),
    )(accp.reshape(NC * N_PAD * C))

    return out1d.reshape(N, C)
